# baseline (speedup 1.0000x reference)
import functools

import jax
import jax.numpy as jnp
from jax import lax
from jax.experimental import pallas as pl
from jax.experimental.pallas import tpu as pltpu
from jax.experimental.pallas import tpu_sc as plsc

_NC = 2
_NS = 16
_LANE = 128
_CHUNK = 128


def _flat_gather(tab_flat, idx_flat):
    n_idx = idx_flat.shape[0]
    nw = _NC * _NS
    per_w = n_idx // nw
    n_chunks = per_w // _CHUNK
    mesh = plsc.VectorSubcoreMesh(core_axis_name="c", subcore_axis_name="s")

    @functools.partial(
        pl.kernel,
        mesh=mesh,
        out_type=jax.ShapeDtypeStruct((n_idx,), jnp.float32),
        scratch_types=[
            pltpu.VMEM((per_w,), jnp.int32),
            pltpu.VMEM((per_w,), jnp.float32),
            pltpu.SemaphoreType.DMA,
        ],
    )
    def k(tab_hbm, idx_hbm, out_hbm, idx_v, vals_v, sem):
        wid = lax.axis_index("s") * _NC + lax.axis_index("c")
        base = wid * per_w
        pltpu.sync_copy(idx_hbm.at[pl.ds(base, per_w)], idx_v)
        half = n_chunks // 2
        for g0 in (0, half):
            cps = [
                pltpu.async_copy(
                    tab_hbm.at[idx_v.at[pl.ds((g0 + j) * _CHUNK, _CHUNK)]],
                    vals_v.at[pl.ds((g0 + j) * _CHUNK, _CHUNK)],
                    sem,
                )
                for j in range(half)
            ]
            for cp in cps:
                cp.wait()
        pltpu.sync_copy(vals_v, out_hbm.at[pl.ds(base, per_w)])

    return k(tab_flat, idx_flat)


def _tc_body(e_ref, g_ref, gt_ref, et_ref, out_ref):
    ti = e_ref.shape[1]
    h_dim = gt_ref.shape[1]
    e = e_ref[0]
    g = g_ref[0]
    gt = gt_ref[0]
    for h in range(h_dim):
        tab = jnp.broadcast_to(et_ref[h : h + 1, :], (ti, _LANE))
        lut = jnp.take_along_axis(tab, e, axis=1)
        gi = g[:, h : h + 1]
        gj = gt[h : h + 1, :]
        out_ref[0, h] = gi * gj + lut


def kernel(node_index, edge_types, graph_table, edge_table):
    b, n = node_index.shape
    v, h_dim = graph_table.shape

    idx = node_index.astype(jnp.int32)
    tab_flat = jnp.transpose(graph_table, (1, 0)).reshape(-1)
    hh = jnp.arange(h_dim, dtype=jnp.int32) * v
    idx_t = (hh.reshape(1, h_dim, 1) + idx.reshape(b, 1, n)).reshape(-1)
    idx_g = (hh.reshape(1, 1, h_dim) + idx.reshape(b, n, 1)).reshape(-1)
    res = _flat_gather(tab_flat, jnp.concatenate([idx_t, idx_g]))
    gt = res[: b * h_dim * n].reshape(b, h_dim, n)
    g3 = res[b * h_dim * n :].reshape(b, n, h_dim)

    et = jnp.zeros((h_dim, _LANE), jnp.float32).at[:, : edge_table.shape[0]].set(
        edge_table.astype(jnp.float32).T
    )

    ti = 128
    grid = (b, n // ti)
    out = pl.pallas_call(
        _tc_body,
        grid=grid,
        in_specs=[
            pl.BlockSpec((1, ti, n), lambda bb, ii: (bb, ii, 0)),
            pl.BlockSpec((1, ti, h_dim), lambda bb, ii: (bb, ii, 0)),
            pl.BlockSpec((1, h_dim, n), lambda bb, ii: (bb, 0, 0)),
            pl.BlockSpec((h_dim, _LANE), lambda bb, ii: (0, 0)),
        ],
        out_specs=pl.BlockSpec((1, h_dim, ti, n), lambda bb, ii: (bb, 0, ii, 0)),
        out_shape=jax.ShapeDtypeStruct((b, h_dim, n, n), jnp.float32),
    )(edge_types, g3, gt, et)
    return out


# speedup vs baseline: 9.2409x; 9.2409x over previous
import functools

import jax
import jax.numpy as jnp
from jax import lax
from jax.experimental import pallas as pl
from jax.experimental.pallas import tpu as pltpu
from jax.experimental.pallas import tpu_sc as plsc

_NC = 2
_NS = 16
_LANE = 128
_CHUNK = 128


def _gather_transposed(tab_t, idx):
    b, n = idx.shape
    h_dim = tab_t.shape[0]
    nw = _NC * _NS
    chunks_per_b = nw // b
    npc = n // chunks_per_b
    mesh = plsc.VectorSubcoreMesh(core_axis_name="c", subcore_axis_name="s")

    n_buf = 8

    @functools.partial(
        pl.kernel,
        mesh=mesh,
        out_type=jax.ShapeDtypeStruct((b, h_dim, n), jnp.float32),
        scratch_types=[
            pltpu.VMEM((npc,), jnp.int32),
            pltpu.VMEM((n_buf, h_dim, _LANE), jnp.float32),
            pltpu.VMEM((h_dim, npc), jnp.float32),
            pltpu.SemaphoreType.DMA,
            pltpu.SemaphoreType.DMA,
        ],
        compiler_params=pltpu.CompilerParams(needs_layout_passes=False),
    )
    def k(tab_hbm, idx_hbm, out_hbm, idx_v, buf_v, vals_v, sem_i, sem):
        wid = lax.axis_index("s") * _NC + lax.axis_index("c")
        bb = wid // chunks_per_b
        q = wid % chunks_per_b
        pltpu.async_copy(idx_hbm.at[bb, pl.ds(q * npc, npc)], idx_v, sem_i).wait()
        row_iota = lax.iota(jnp.int32, h_dim)
        lane = h_dim

        def node_scalar(j):
            chunk = idx_v[pl.ds((j // lane) * lane, lane)]
            sel = jnp.where(row_iota == (j % lane), chunk, 0)
            return jnp.sum(sel)

        def tile_base(v):
            return pl.multiple_of((v // _LANE) * _LANE, _LANE)

        vs = [None] * npc

        def fetch(j):
            vs[j] = node_scalar(j)
            return pltpu.async_copy(
                tab_hbm.at[:, pl.ds(tile_base(vs[j]), _LANE)],
                buf_v.at[j % n_buf],
                sem,
            )

        cps = [fetch(j) for j in range(n_buf)]
        for j in range(npc):
            cps[j % n_buf].wait()
            off = vs[j] - tile_base(vs[j])
            col = plsc.load_gather(
                buf_v.at[j % n_buf],
                [row_iota, jnp.full((h_dim,), 0, jnp.int32) + off],
            )
            plsc.store_scatter(
                vals_v,
                [row_iota, jnp.full((h_dim,), j, jnp.int32)],
                col,
            )
            if j + n_buf < npc:
                cps[j % n_buf] = fetch(j + n_buf)
        pltpu.sync_copy(vals_v, out_hbm.at[bb].at[:, pl.ds(q * npc, npc)])

    return k(tab_t, idx)


def _tc_body(e_ref, gt_ref, gti_ref, et_ref, out_ref):
    ti = e_ref.shape[1]
    h_dim = gt_ref.shape[1]
    e = e_ref[0]
    gt = gt_ref[0]
    gi_all = jnp.transpose(gti_ref[0], (1, 0))
    for h in range(h_dim):
        tab = jnp.broadcast_to(et_ref[h : h + 1, :], (ti, _LANE))
        lut = jnp.take_along_axis(tab, e, axis=1)
        gi = gi_all[:, h : h + 1]
        gj = gt[h : h + 1, :]
        out_ref[0, h] = gi * gj + lut


def kernel(node_index, edge_types, graph_table, edge_table):
    b, n = node_index.shape
    v, h_dim = graph_table.shape

    idx = node_index.astype(jnp.int32)
    tab_t = jnp.transpose(graph_table, (1, 0))
    gt = _gather_transposed(tab_t, idx)

    et = jnp.zeros((h_dim, _LANE), jnp.float32).at[:, : edge_table.shape[0]].set(
        edge_table.astype(jnp.float32).T
    )

    ti = 128
    grid = (b, n // ti)
    out = pl.pallas_call(
        _tc_body,
        grid=grid,
        in_specs=[
            pl.BlockSpec((1, ti, n), lambda bb, ii: (bb, ii, 0)),
            pl.BlockSpec((1, h_dim, n), lambda bb, ii: (bb, 0, 0)),
            pl.BlockSpec((1, h_dim, ti), lambda bb, ii: (bb, 0, ii)),
            pl.BlockSpec((h_dim, _LANE), lambda bb, ii: (0, 0)),
        ],
        out_specs=pl.BlockSpec((1, h_dim, ti, n), lambda bb, ii: (bb, 0, ii, 0)),
        out_shape=jax.ShapeDtypeStruct((b, h_dim, n, n), jnp.float32),
    )(edge_types, gt, gt, et)
    return out


# speedup vs baseline: 12.3841x; 1.3401x over previous
import functools

import jax
import jax.numpy as jnp
from jax import lax
from jax.experimental import pallas as pl
from jax.experimental.pallas import tpu as pltpu
from jax.experimental.pallas import tpu_sc as plsc

_NC = 2
_NS = 16
_LANE = 128
_CHUNK = 128


def _gather_transposed(tab_t, idx):
    b, n = idx.shape
    h_dim = tab_t.shape[0]
    nw = _NC * _NS
    chunks_per_b = nw // b
    npc = n // chunks_per_b
    mesh = plsc.VectorSubcoreMesh(core_axis_name="c", subcore_axis_name="s")

    n_buf = 8

    @functools.partial(
        pl.kernel,
        mesh=mesh,
        out_type=jax.ShapeDtypeStruct((b, h_dim, n), jnp.float32),
        scratch_types=[
            pltpu.VMEM((npc,), jnp.int32),
            pltpu.VMEM((n_buf, h_dim, _LANE), jnp.float32),
            pltpu.VMEM((h_dim, npc), jnp.float32),
            pltpu.SemaphoreType.DMA,
            pltpu.SemaphoreType.DMA,
        ],
        compiler_params=pltpu.CompilerParams(needs_layout_passes=False),
    )
    def k(tab_hbm, idx_hbm, out_hbm, idx_v, buf_v, vals_v, sem_i, sem):
        wid = lax.axis_index("s") * _NC + lax.axis_index("c")
        bb = wid // chunks_per_b
        q = wid % chunks_per_b
        pltpu.async_copy(idx_hbm.at[bb, pl.ds(q * npc, npc)], idx_v, sem_i).wait()
        row_iota = lax.iota(jnp.int32, h_dim)
        lane = h_dim

        def node_scalar(j):
            chunk = idx_v[pl.ds((j // lane) * lane, lane)]
            sel = jnp.where(row_iota == (j % lane), chunk, 0)
            return jnp.sum(sel)

        def tile_base(v):
            return pl.multiple_of((v // _LANE) * _LANE, _LANE)

        vs = [None] * npc

        def fetch(j):
            vs[j] = node_scalar(j)
            return pltpu.async_copy(
                tab_hbm.at[:, pl.ds(tile_base(vs[j]), _LANE)],
                buf_v.at[j % n_buf],
                sem,
            )

        cps = [fetch(j) for j in range(n_buf)]
        for j in range(npc):
            cps[j % n_buf].wait()
            off = vs[j] - tile_base(vs[j])
            col = plsc.load_gather(
                buf_v.at[j % n_buf],
                [row_iota, jnp.full((h_dim,), 0, jnp.int32) + off],
            )
            plsc.store_scatter(
                vals_v,
                [row_iota, jnp.full((h_dim,), j, jnp.int32)],
                col,
            )
            if j + n_buf < npc:
                cps[j % n_buf] = fetch(j + n_buf)
        pltpu.sync_copy(vals_v, out_hbm.at[bb].at[:, pl.ds(q * npc, npc)])

    return k(tab_t, idx)


def _tc_body(e_ref, gt_ref, gti_ref, et_ref, out_ref):
    ti = e_ref.shape[1]
    n = e_ref.shape[2]
    h_dim = gt_ref.shape[1]
    gt = gt_ref[0]
    gi_all = jnp.transpose(gti_ref[0], (1, 0))
    e = e_ref[0]
    tabs = [
        jnp.broadcast_to(et_ref[h : h + 1, :], (8, _LANE)) for h in range(h_dim)
    ]
    for jc in range(n // _LANE):
        j0, j1 = jc * _LANE, (jc + 1) * _LANE
        for ir in range(ti // 8):
            i0, i1 = ir * 8, (ir + 1) * 8
            e_v = e[i0:i1, j0:j1]
            for h in range(h_dim):
                lut = jnp.take_along_axis(tabs[h], e_v, axis=1)
                gi = gi_all[i0:i1, h : h + 1]
                gj = gt[h : h + 1, j0:j1]
                out_ref[0, h, i0:i1, j0:j1] = gi * gj + lut


def kernel(node_index, edge_types, graph_table, edge_table):
    b, n = node_index.shape
    v, h_dim = graph_table.shape

    idx = node_index.astype(jnp.int32)
    tab_t = jnp.transpose(graph_table, (1, 0))
    gt = _gather_transposed(tab_t, idx)

    et = jnp.zeros((h_dim, _LANE), jnp.float32).at[:, : edge_table.shape[0]].set(
        edge_table.astype(jnp.float32).T
    )

    ti = 128
    grid = (b, n // ti)
    out = pl.pallas_call(
        _tc_body,
        grid=grid,
        in_specs=[
            pl.BlockSpec((1, ti, n), lambda bb, ii: (bb, ii, 0)),
            pl.BlockSpec((1, h_dim, n), lambda bb, ii: (bb, 0, 0)),
            pl.BlockSpec((1, h_dim, ti), lambda bb, ii: (bb, 0, ii)),
            pl.BlockSpec((h_dim, _LANE), lambda bb, ii: (0, 0)),
        ],
        out_specs=pl.BlockSpec((1, h_dim, ti, n), lambda bb, ii: (bb, 0, ii, 0)),
        out_shape=jax.ShapeDtypeStruct((b, h_dim, n, n), jnp.float32),
    )(edge_types, gt, gt, et)
    return out


# speedup vs baseline: 12.3926x; 1.0007x over previous
import functools

import jax
import jax.numpy as jnp
from jax import lax
from jax.experimental import pallas as pl
from jax.experimental.pallas import tpu as pltpu
from jax.experimental.pallas import tpu_sc as plsc

_NC = 2
_NS = 16
_LANE = 128
_CHUNK = 128


def _gather_transposed(tab_t, idx):
    b, n = idx.shape
    h_dim = tab_t.shape[0]
    nw = _NC * _NS
    chunks_per_b = nw // b
    npc = n // chunks_per_b
    mesh = plsc.VectorSubcoreMesh(core_axis_name="c", subcore_axis_name="s")

    n_buf = 8

    @functools.partial(
        pl.kernel,
        mesh=mesh,
        out_type=jax.ShapeDtypeStruct((b, h_dim, n), jnp.float32),
        scratch_types=[
            pltpu.VMEM((npc,), jnp.int32),
            pltpu.VMEM((n_buf, h_dim, _LANE), jnp.float32),
            pltpu.VMEM((h_dim, npc), jnp.float32),
            pltpu.SemaphoreType.DMA,
            pltpu.SemaphoreType.DMA,
        ],
        compiler_params=pltpu.CompilerParams(needs_layout_passes=False),
    )
    def k(tab_hbm, idx_hbm, out_hbm, idx_v, buf_v, vals_v, sem_i, sem):
        wid = lax.axis_index("s") * _NC + lax.axis_index("c")
        bb = wid // chunks_per_b
        q = wid % chunks_per_b
        pltpu.async_copy(idx_hbm.at[bb, pl.ds(q * npc, npc)], idx_v, sem_i).wait()
        row_iota = lax.iota(jnp.int32, h_dim)
        lane = h_dim

        def node_scalar(j):
            chunk = idx_v[pl.ds((j // lane) * lane, lane)]
            sel = jnp.where(row_iota == (j % lane), chunk, 0)
            return jnp.sum(sel)

        def tile_base(v):
            return pl.multiple_of((v // _LANE) * _LANE, _LANE)

        vs = [None] * npc

        def fetch(j):
            vs[j] = node_scalar(j)
            return pltpu.async_copy(
                tab_hbm.at[:, pl.ds(tile_base(vs[j]), _LANE)],
                buf_v.at[j % n_buf],
                sem,
            )

        cps = [fetch(j) for j in range(n_buf)]
        for j in range(npc):
            cps[j % n_buf].wait()
            off = vs[j] - tile_base(vs[j])
            col = plsc.load_gather(
                buf_v.at[j % n_buf],
                [row_iota, jnp.full((h_dim,), 0, jnp.int32) + off],
            )
            plsc.store_scatter(
                vals_v,
                [row_iota, jnp.full((h_dim,), j, jnp.int32)],
                col,
            )
            if j + n_buf < npc:
                cps[j % n_buf] = fetch(j + n_buf)
        pltpu.sync_copy(vals_v, out_hbm.at[bb].at[:, pl.ds(q * npc, npc)])

    return k(tab_t, idx)


def _tc_body(e_ref, gt_ref, gti_ref, et_ref, out_ref):
    ti = e_ref.shape[1]
    n = e_ref.shape[2]
    h_dim = gt_ref.shape[1]
    gt = gt_ref[0]
    gi_all = jnp.transpose(gti_ref[0], (1, 0))
    e = e_ref[0]
    tabs = [
        jnp.broadcast_to(et_ref[h : h + 1, :], (8, _LANE)) for h in range(h_dim)
    ]
    for jc in range(n // _LANE):
        j0, j1 = jc * _LANE, (jc + 1) * _LANE
        for ir in range(ti // 8):
            i0, i1 = ir * 8, (ir + 1) * 8
            e_v = e[i0:i1, j0:j1]
            for h in range(h_dim):
                lut = jnp.take_along_axis(tabs[h], e_v, axis=1)
                gi = gi_all[i0:i1, h : h + 1]
                gj = gt[h : h + 1, j0:j1]
                out_ref[0, h, i0:i1, j0:j1] = gi * gj + lut


def kernel(node_index, edge_types, graph_table, edge_table):
    b, n = node_index.shape
    v, h_dim = graph_table.shape

    idx = node_index.astype(jnp.int32)
    tab_t = jnp.transpose(graph_table, (1, 0))
    gt = _gather_transposed(tab_t, idx)

    et = jnp.zeros((h_dim, _LANE), jnp.float32).at[:, : edge_table.shape[0]].set(
        edge_table.astype(jnp.float32).T
    )

    ti = 128
    grid = (b, n // ti)
    out = pl.pallas_call(
        _tc_body,
        grid=grid,
        in_specs=[
            pl.BlockSpec((1, ti, n), lambda bb, ii: (bb, ii, 0)),
            pl.BlockSpec((1, h_dim, n), lambda bb, ii: (bb, 0, 0)),
            pl.BlockSpec((1, h_dim, ti), lambda bb, ii: (bb, 0, ii)),
            pl.BlockSpec((h_dim, _LANE), lambda bb, ii: (0, 0)),
        ],
        out_specs=pl.BlockSpec((1, h_dim, ti, n), lambda bb, ii: (bb, 0, ii, 0)),
        out_shape=jax.ShapeDtypeStruct((b, h_dim, n, n), jnp.float32),
        compiler_params=pltpu.CompilerParams(
            dimension_semantics=("parallel", "parallel")
        ),
    )(edge_types, gt, gt, et)
    return out


# speedup vs baseline: 13.0643x; 1.0542x over previous
import functools

import jax
import jax.numpy as jnp
from jax import lax
from jax.experimental import pallas as pl
from jax.experimental.pallas import tpu as pltpu
from jax.experimental.pallas import tpu_sc as plsc

_NC = 2
_NS = 16
_LANE = 128
_CHUNK = 128


def _gather_transposed(tab_t, idx):
    b, n = idx.shape
    h_dim = tab_t.shape[0]
    nw = _NC * _NS
    chunks_per_b = nw // b
    npc = n // chunks_per_b
    mesh = plsc.VectorSubcoreMesh(core_axis_name="c", subcore_axis_name="s")

    n_buf = 8

    @functools.partial(
        pl.kernel,
        mesh=mesh,
        out_type=jax.ShapeDtypeStruct((b, h_dim, n), jnp.float32),
        scratch_types=[
            pltpu.VMEM((npc,), jnp.int32),
            pltpu.VMEM((n_buf, h_dim, _LANE), jnp.float32),
            pltpu.VMEM((h_dim, npc), jnp.float32),
            pltpu.SemaphoreType.DMA,
            pltpu.SemaphoreType.DMA,
        ],
        compiler_params=pltpu.CompilerParams(needs_layout_passes=False),
    )
    def k(tab_hbm, idx_hbm, out_hbm, idx_v, buf_v, vals_v, sem_i, sem):
        wid = lax.axis_index("s") * _NC + lax.axis_index("c")
        bb = wid // chunks_per_b
        q = wid % chunks_per_b
        pltpu.async_copy(idx_hbm.at[bb, pl.ds(q * npc, npc)], idx_v, sem_i).wait()
        row_iota = lax.iota(jnp.int32, h_dim)
        lane = h_dim

        def node_scalar(j):
            chunk = idx_v[pl.ds((j // lane) * lane, lane)]
            sel = jnp.where(row_iota == (j % lane), chunk, 0)
            return jnp.sum(sel)

        def tile_base(v):
            return pl.multiple_of((v // _LANE) * _LANE, _LANE)

        vs = [None] * npc

        def fetch(j):
            vs[j] = node_scalar(j)
            return pltpu.async_copy(
                tab_hbm.at[:, pl.ds(tile_base(vs[j]), _LANE)],
                buf_v.at[j % n_buf],
                sem,
            )

        cps = [fetch(j) for j in range(n_buf)]
        for j in range(npc):
            cps[j % n_buf].wait()
            off = vs[j] - tile_base(vs[j])
            col = plsc.load_gather(
                buf_v.at[j % n_buf],
                [row_iota, jnp.full((h_dim,), 0, jnp.int32) + off],
            )
            plsc.store_scatter(
                vals_v,
                [row_iota, jnp.full((h_dim,), j, jnp.int32)],
                col,
            )
            if j + n_buf < npc:
                cps[j % n_buf] = fetch(j + n_buf)
        pltpu.sync_copy(vals_v, out_hbm.at[bb].at[:, pl.ds(q * npc, npc)])

    return k(tab_t, idx)


def _tc_body(e_ref, gt_ref, gti_ref, et_ref, out_ref):
    ti = e_ref.shape[1]
    n = e_ref.shape[2]
    h_dim = gt_ref.shape[1]
    gt = gt_ref[0]
    gi_all = jnp.transpose(gti_ref[0], (1, 0))
    e = e_ref[0]
    tabs = [
        jnp.broadcast_to(et_ref[h : h + 1, :], (8, _LANE)) for h in range(h_dim)
    ]
    for ir in range(ti // 8):
        i0, i1 = ir * 8, (ir + 1) * 8
        gi_b = [
            jnp.broadcast_to(gi_all[i0:i1, h : h + 1], (8, _LANE))
            for h in range(h_dim)
        ]
        for jc in range(n // _LANE):
            j0, j1 = jc * _LANE, (jc + 1) * _LANE
            e_v = e[i0:i1, j0:j1]
            for h in range(h_dim):
                lut = jnp.take_along_axis(tabs[h], e_v, axis=1)
                gj = gt[h : h + 1, j0:j1]
                out_ref[0, h, i0:i1, j0:j1] = gi_b[h] * gj + lut


def kernel(node_index, edge_types, graph_table, edge_table):
    b, n = node_index.shape
    v, h_dim = graph_table.shape

    idx = node_index.astype(jnp.int32)
    tab_t = jnp.transpose(graph_table, (1, 0))
    gt = _gather_transposed(tab_t, idx)

    et = jnp.zeros((h_dim, _LANE), jnp.float32).at[:, : edge_table.shape[0]].set(
        edge_table.astype(jnp.float32).T
    )

    ti = 128
    grid = (b, n // ti)
    out = pl.pallas_call(
        _tc_body,
        grid=grid,
        in_specs=[
            pl.BlockSpec((1, ti, n), lambda bb, ii: (bb, ii, 0)),
            pl.BlockSpec((1, h_dim, n), lambda bb, ii: (bb, 0, 0)),
            pl.BlockSpec((1, h_dim, ti), lambda bb, ii: (bb, 0, ii)),
            pl.BlockSpec((h_dim, _LANE), lambda bb, ii: (0, 0)),
        ],
        out_specs=pl.BlockSpec((1, h_dim, ti, n), lambda bb, ii: (bb, 0, ii, 0)),
        out_shape=jax.ShapeDtypeStruct((b, h_dim, n, n), jnp.float32),
        compiler_params=pltpu.CompilerParams(
            dimension_semantics=("parallel", "parallel")
        ),
    )(edge_types, gt, gt, et)
    return out


# speedup vs baseline: 13.9854x; 1.0705x over previous
import functools

import jax
import jax.numpy as jnp
from jax import lax
from jax.experimental import pallas as pl
from jax.experimental.pallas import tpu as pltpu
from jax.experimental.pallas import tpu_sc as plsc

_NC = 2
_NS = 16
_LANE = 128
_CHUNK = 128


def _gather_transposed(tab_t, idx):
    b, n = idx.shape
    h_dim = tab_t.shape[0]
    nw = _NC * _NS
    chunks_per_b = nw // b
    npc = n // chunks_per_b
    mesh = plsc.VectorSubcoreMesh(core_axis_name="c", subcore_axis_name="s")

    n_buf = 8

    @functools.partial(
        pl.kernel,
        mesh=mesh,
        out_type=jax.ShapeDtypeStruct((b, h_dim, n), jnp.float32),
        scratch_types=[
            pltpu.VMEM((npc,), jnp.int32),
            pltpu.VMEM((n_buf, h_dim, _LANE), jnp.float32),
            pltpu.VMEM((h_dim, npc), jnp.float32),
            pltpu.SemaphoreType.DMA,
            pltpu.SemaphoreType.DMA,
        ],
        compiler_params=pltpu.CompilerParams(needs_layout_passes=False),
    )
    def k(tab_hbm, idx_hbm, out_hbm, idx_v, buf_v, vals_v, sem_i, sem):
        wid = lax.axis_index("s") * _NC + lax.axis_index("c")
        bb = wid // chunks_per_b
        q = wid % chunks_per_b
        pltpu.async_copy(idx_hbm.at[bb, pl.ds(q * npc, npc)], idx_v, sem_i).wait()
        row_iota = lax.iota(jnp.int32, h_dim)
        lane = h_dim

        def node_scalar(j):
            chunk = idx_v[pl.ds((j // lane) * lane, lane)]
            sel = jnp.where(row_iota == (j % lane), chunk, 0)
            return jnp.sum(sel)

        def tile_base(v):
            return pl.multiple_of((v // _LANE) * _LANE, _LANE)

        vs = [None] * npc

        def fetch(j):
            vs[j] = node_scalar(j)
            return pltpu.async_copy(
                tab_hbm.at[:, pl.ds(tile_base(vs[j]), _LANE)],
                buf_v.at[j % n_buf],
                sem,
            )

        cps = [fetch(j) for j in range(n_buf)]
        for j in range(npc):
            cps[j % n_buf].wait()
            off = vs[j] - tile_base(vs[j])
            col = plsc.load_gather(
                buf_v.at[j % n_buf],
                [row_iota, jnp.full((h_dim,), 0, jnp.int32) + off],
            )
            plsc.store_scatter(
                vals_v,
                [row_iota, jnp.full((h_dim,), j, jnp.int32)],
                col,
            )
            if j + n_buf < npc:
                cps[j % n_buf] = fetch(j + n_buf)
        pltpu.sync_copy(vals_v, out_hbm.at[bb].at[:, pl.ds(q * npc, npc)])

    return k(tab_t, idx)


def _tc_body(e_ref, gt_ref, gti_ref, etp_ref, out_ref):
    ti = e_ref.shape[1]
    n = e_ref.shape[2]
    h_dim = gt_ref.shape[1]
    gt = gt_ref[0]
    gi_all = jnp.transpose(gti_ref[0], (1, 0))
    e = e_ref[0]
    tabs = [
        jnp.broadcast_to(etp_ref[p : p + 1, :], (8, _LANE))
        for p in range(h_dim // 2)
    ]
    hi_mask = jnp.int32(-65536)
    for ir in range(ti // 8):
        i0, i1 = ir * 8, (ir + 1) * 8
        gi_b = [
            jnp.broadcast_to(gi_all[i0:i1, h : h + 1], (8, _LANE))
            for h in range(h_dim)
        ]
        for jc in range(n // _LANE):
            j0, j1 = jc * _LANE, (jc + 1) * _LANE
            e_v = e[i0:i1, j0:j1]
            for p in range(h_dim // 2):
                ha, hb = 2 * p, 2 * p + 1
                lutp = jnp.take_along_axis(tabs[p], e_v, axis=1)
                fa = lax.bitcast_convert_type(lutp & hi_mask, jnp.float32)
                fb = lax.bitcast_convert_type(
                    lax.shift_left(lutp, jnp.int32(16)), jnp.float32
                )
                out_ref[0, ha, i0:i1, j0:j1] = (
                    gi_b[ha] * gt[ha : ha + 1, j0:j1] + fa
                )
                out_ref[0, hb, i0:i1, j0:j1] = (
                    gi_b[hb] * gt[hb : hb + 1, j0:j1] + fb
                )


def kernel(node_index, edge_types, graph_table, edge_table):
    b, n = node_index.shape
    v, h_dim = graph_table.shape

    idx = node_index.astype(jnp.int32)
    tab_t = jnp.transpose(graph_table, (1, 0))
    gt = _gather_transposed(tab_t, idx)

    et = jnp.zeros((h_dim, _LANE), jnp.float32).at[:, : edge_table.shape[0]].set(
        edge_table.astype(jnp.float32).T
    )
    u = lax.bitcast_convert_type(et, jnp.uint32) + jnp.uint32(0x8000)
    etp = lax.bitcast_convert_type(
        (u[0::2] & jnp.uint32(0xFFFF0000)) | (u[1::2] >> 16), jnp.int32
    )

    ti = 128
    grid = (b, n // ti)
    out = pl.pallas_call(
        _tc_body,
        grid=grid,
        in_specs=[
            pl.BlockSpec((1, ti, n), lambda bb, ii: (bb, ii, 0)),
            pl.BlockSpec((1, h_dim, n), lambda bb, ii: (bb, 0, 0)),
            pl.BlockSpec((1, h_dim, ti), lambda bb, ii: (bb, 0, ii)),
            pl.BlockSpec((h_dim // 2, _LANE), lambda bb, ii: (0, 0)),
        ],
        out_specs=pl.BlockSpec((1, h_dim, ti, n), lambda bb, ii: (bb, 0, ii, 0)),
        out_shape=jax.ShapeDtypeStruct((b, h_dim, n, n), jnp.float32),
        compiler_params=pltpu.CompilerParams(
            dimension_semantics=("parallel", "parallel")
        ),
    )(edge_types, gt, gt, etp)
    return out


# speedup vs baseline: 14.4881x; 1.0359x over previous
import functools

import jax
import jax.numpy as jnp
from jax import lax
from jax.experimental import pallas as pl
from jax.experimental.pallas import tpu as pltpu
from jax.experimental.pallas import tpu_sc as plsc

_NC = 2
_NS = 16
_LANE = 128
_CHUNK = 128


def _gather_transposed(tab_t, idx):
    b, n = idx.shape
    h_dim = tab_t.shape[0]
    nw = _NC * _NS
    chunks_per_b = nw // b
    npc = n // chunks_per_b
    mesh = plsc.VectorSubcoreMesh(core_axis_name="c", subcore_axis_name="s")

    n_buf = 16

    @functools.partial(
        pl.kernel,
        mesh=mesh,
        out_type=jax.ShapeDtypeStruct((b, h_dim, n), jnp.float32),
        scratch_types=[
            pltpu.VMEM((npc,), jnp.int32),
            pltpu.VMEM((n_buf, h_dim, _LANE), jnp.float32),
            pltpu.VMEM((h_dim, npc), jnp.float32),
            pltpu.SemaphoreType.DMA,
            pltpu.SemaphoreType.DMA,
        ],
        compiler_params=pltpu.CompilerParams(needs_layout_passes=False),
    )
    def k(tab_hbm, idx_hbm, out_hbm, idx_v, buf_v, vals_v, sem_i, sem):
        wid = lax.axis_index("s") * _NC + lax.axis_index("c")
        bb = wid // chunks_per_b
        q = wid % chunks_per_b
        pltpu.async_copy(idx_hbm.at[bb, pl.ds(q * npc, npc)], idx_v, sem_i).wait()
        row_iota = lax.iota(jnp.int32, h_dim)
        lane = h_dim

        def node_scalar(j):
            chunk = idx_v[pl.ds((j // lane) * lane, lane)]
            sel = jnp.where(row_iota == (j % lane), chunk, 0)
            return jnp.sum(sel)

        def tile_base(v):
            return pl.multiple_of((v // _LANE) * _LANE, _LANE)

        vs = [None] * npc

        def fetch(j):
            vs[j] = node_scalar(j)
            return pltpu.async_copy(
                tab_hbm.at[:, pl.ds(tile_base(vs[j]), _LANE)],
                buf_v.at[j % n_buf],
                sem,
            )

        cps = [fetch(j) for j in range(n_buf)]
        for j in range(npc):
            cps[j % n_buf].wait()
            off = vs[j] - tile_base(vs[j])
            col = plsc.load_gather(
                buf_v.at[j % n_buf],
                [row_iota, jnp.full((h_dim,), 0, jnp.int32) + off],
            )
            plsc.store_scatter(
                vals_v,
                [row_iota, jnp.full((h_dim,), j, jnp.int32)],
                col,
            )
            if j + n_buf < npc:
                cps[j % n_buf] = fetch(j + n_buf)
        pltpu.sync_copy(vals_v, out_hbm.at[bb].at[:, pl.ds(q * npc, npc)])

    return k(tab_t, idx)


def _tc_body(e_ref, gt_ref, gti_ref, etp_ref, out_ref):
    ti = e_ref.shape[1]
    n = e_ref.shape[2]
    h_dim = gt_ref.shape[1]
    gt = gt_ref[0]
    gi_all = jnp.transpose(gti_ref[0], (1, 0))
    e = e_ref[0]
    tabs = [
        jnp.broadcast_to(etp_ref[p : p + 1, :], (8, _LANE))
        for p in range(h_dim // 2)
    ]
    hi_mask = jnp.int32(-65536)
    for ir in range(ti // 8):
        i0, i1 = ir * 8, (ir + 1) * 8
        gi_b = [
            jnp.broadcast_to(gi_all[i0:i1, h : h + 1], (8, _LANE))
            for h in range(h_dim)
        ]
        for jc in range(n // _LANE):
            j0, j1 = jc * _LANE, (jc + 1) * _LANE
            e_v = e[i0:i1, j0:j1]
            for p in range(h_dim // 2):
                ha, hb = 2 * p, 2 * p + 1
                lutp = jnp.take_along_axis(tabs[p], e_v, axis=1)
                fa = lax.bitcast_convert_type(lutp & hi_mask, jnp.float32)
                fb = lax.bitcast_convert_type(
                    lax.shift_left(lutp, jnp.int32(16)), jnp.float32
                )
                out_ref[0, ha, i0:i1, j0:j1] = (
                    gi_b[ha] * gt[ha : ha + 1, j0:j1] + fa
                )
                out_ref[0, hb, i0:i1, j0:j1] = (
                    gi_b[hb] * gt[hb : hb + 1, j0:j1] + fb
                )


def kernel(node_index, edge_types, graph_table, edge_table):
    b, n = node_index.shape
    v, h_dim = graph_table.shape

    idx = node_index.astype(jnp.int32)
    tab_t = jnp.transpose(graph_table, (1, 0))
    gt = _gather_transposed(tab_t, idx)

    et = jnp.zeros((h_dim, _LANE), jnp.float32).at[:, : edge_table.shape[0]].set(
        edge_table.astype(jnp.float32).T
    )
    u = lax.bitcast_convert_type(et, jnp.uint32) + jnp.uint32(0x8000)
    etp = lax.bitcast_convert_type(
        (u[0::2] & jnp.uint32(0xFFFF0000)) | (u[1::2] >> 16), jnp.int32
    )

    ti = 128
    grid = (b, n // ti)
    out = pl.pallas_call(
        _tc_body,
        grid=grid,
        in_specs=[
            pl.BlockSpec((1, ti, n), lambda bb, ii: (bb, ii, 0)),
            pl.BlockSpec((1, h_dim, n), lambda bb, ii: (bb, 0, 0)),
            pl.BlockSpec((1, h_dim, ti), lambda bb, ii: (bb, 0, ii)),
            pl.BlockSpec((h_dim // 2, _LANE), lambda bb, ii: (0, 0)),
        ],
        out_specs=pl.BlockSpec((1, h_dim, ti, n), lambda bb, ii: (bb, 0, ii, 0)),
        out_shape=jax.ShapeDtypeStruct((b, h_dim, n, n), jnp.float32),
        compiler_params=pltpu.CompilerParams(
            dimension_semantics=("parallel", "parallel")
        ),
    )(edge_types, gt, gt, etp)
    return out
